# baseline (device time: 31089 ns/iter reference)
import jax
import jax.numpy as jnp
from jax import lax
from jax.experimental import pallas as pl
from jax.experimental.pallas import tpu as pltpu

N_DEV = 16


def kernel(A, B):
    m, k = A.shape
    _, n = B.shape
    chunk = m // N_DEV

    def body(a_ref, b_ref, out_ref, part_ref, rs_ref, bc_ref,
             s1_sems, r1_sems, s2_sems, r2_sems):
        my = lax.axis_index("i")

        barrier = pltpu.get_barrier_semaphore()
        for s in range(1, N_DEV):
            dst = lax.rem(my + s, N_DEV)
            pl.semaphore_signal(
                barrier, inc=1, device_id=(dst,),
                device_id_type=pl.DeviceIdType.MESH,
            )
        pl.semaphore_wait(barrier, N_DEV - 1)

        a = a_ref[...].astype(jnp.bfloat16)
        b = b_ref[...].astype(jnp.bfloat16)
        part_ref[...] = jnp.dot(a, b, preferred_element_type=jnp.float32)

        rs_ref[0] = part_ref[pl.ds(my * chunk, chunk), :]

        p1 = []
        for s in range(1, N_DEV):
            dst = lax.rem(my + s, N_DEV)
            rdma = pltpu.make_async_remote_copy(
                src_ref=part_ref.at[pl.ds(dst * chunk, chunk), :],
                dst_ref=rs_ref.at[s],
                send_sem=s1_sems.at[s],
                recv_sem=r1_sems.at[s],
                device_id=(dst,),
                device_id_type=pl.DeviceIdType.MESH,
            )
            rdma.start()
            p1.append(rdma)
        for rdma in p1:
            rdma.wait_recv()

        reduced = jnp.maximum(jnp.sum(rs_ref[...], axis=0), 0.0)
        bc_ref[...] = reduced
        out_ref[pl.ds(my * chunk, chunk), :] = reduced

        p2 = []
        for s in range(1, N_DEV):
            dst = lax.rem(my + s, N_DEV)
            rdma = pltpu.make_async_remote_copy(
                src_ref=bc_ref,
                dst_ref=out_ref.at[pl.ds(my * chunk, chunk), :],
                send_sem=s2_sems.at[s],
                recv_sem=r2_sems.at[s],
                device_id=(dst,),
                device_id_type=pl.DeviceIdType.MESH,
            )
            rdma.start()
            p2.append(rdma)
        for rdma in p2:
            rdma.wait_recv()
        for rdma in p1:
            rdma.wait_send()
        for rdma in p2:
            rdma.wait_send()

    return pl.pallas_call(
        body,
        out_shape=jax.ShapeDtypeStruct((m, n), jnp.float32),
        in_specs=[
            pl.BlockSpec(memory_space=pltpu.VMEM),
            pl.BlockSpec(memory_space=pltpu.VMEM),
        ],
        out_specs=pl.BlockSpec(memory_space=pltpu.VMEM),
        scratch_shapes=[
            pltpu.VMEM((m, n), jnp.float32),
            pltpu.VMEM((N_DEV, chunk, n), jnp.float32),
            pltpu.VMEM((chunk, n), jnp.float32),
            pltpu.SemaphoreType.DMA((N_DEV,)),
            pltpu.SemaphoreType.DMA((N_DEV,)),
            pltpu.SemaphoreType.DMA((N_DEV,)),
            pltpu.SemaphoreType.DMA((N_DEV,)),
        ],
        compiler_params=pltpu.CompilerParams(collective_id=0),
    )(A, B)


# device time: 22639 ns/iter; 1.3732x vs baseline; 1.3732x over previous
import jax
import jax.numpy as jnp
from jax import lax
from jax.experimental import pallas as pl
from jax.experimental.pallas import tpu as pltpu

N_DEV = 16


def kernel(A, B):
    m, k = A.shape
    _, n = B.shape
    chunk = m // N_DEV

    def body(a_ref, b_ref, out_ref, part_ref, rs_ref, bc_ref,
             s1_sems, r1_sems, s2_sems, r2_sems):
        my = lax.axis_index("i")

        barrier = pltpu.get_barrier_semaphore()
        for s in range(1, N_DEV):
            dst = lax.rem(my + s, N_DEV)
            pl.semaphore_signal(
                barrier, inc=1, device_id=(dst,),
                device_id_type=pl.DeviceIdType.MESH,
            )
        pl.semaphore_wait(barrier, N_DEV - 1)

        a = a_ref[...].astype(jnp.bfloat16)
        b = b_ref[...].astype(jnp.bfloat16)
        part_ref[...] = jnp.dot(
            a, b, preferred_element_type=jnp.float32
        ).astype(jnp.bfloat16)

        rs_ref[0] = part_ref[pl.ds(my * chunk, chunk), :]

        p1 = []
        for s in range(1, N_DEV):
            dst = lax.rem(my + s, N_DEV)
            rdma = pltpu.make_async_remote_copy(
                src_ref=part_ref.at[pl.ds(dst * chunk, chunk), :],
                dst_ref=rs_ref.at[s],
                send_sem=s1_sems.at[s],
                recv_sem=r1_sems.at[s],
                device_id=(dst,),
                device_id_type=pl.DeviceIdType.MESH,
            )
            rdma.start()
            p1.append(rdma)
        for rdma in p1:
            rdma.wait_recv()

        reduced = jnp.maximum(
            jnp.sum(rs_ref[...].astype(jnp.float32), axis=0), 0.0
        ).astype(jnp.bfloat16)
        bc_ref[...] = reduced
        out_ref[pl.ds(my * chunk, chunk), :] = reduced

        p2 = []
        for s in range(1, N_DEV):
            dst = lax.rem(my + s, N_DEV)
            rdma = pltpu.make_async_remote_copy(
                src_ref=bc_ref,
                dst_ref=out_ref.at[pl.ds(my * chunk, chunk), :],
                send_sem=s2_sems.at[s],
                recv_sem=r2_sems.at[s],
                device_id=(dst,),
                device_id_type=pl.DeviceIdType.MESH,
            )
            rdma.start()
            p2.append(rdma)
        for rdma in p2:
            rdma.wait_recv()
        for rdma in p1:
            rdma.wait_send()
        for rdma in p2:
            rdma.wait_send()

    return pl.pallas_call(
        body,
        out_shape=jax.ShapeDtypeStruct((m, n), jnp.bfloat16),
        in_specs=[
            pl.BlockSpec(memory_space=pltpu.VMEM),
            pl.BlockSpec(memory_space=pltpu.VMEM),
        ],
        out_specs=pl.BlockSpec(memory_space=pltpu.VMEM),
        scratch_shapes=[
            pltpu.VMEM((m, n), jnp.bfloat16),
            pltpu.VMEM((N_DEV, chunk, n), jnp.bfloat16),
            pltpu.VMEM((chunk, n), jnp.bfloat16),
            pltpu.SemaphoreType.DMA((N_DEV,)),
            pltpu.SemaphoreType.DMA((N_DEV,)),
            pltpu.SemaphoreType.DMA((N_DEV,)),
            pltpu.SemaphoreType.DMA((N_DEV,)),
        ],
        compiler_params=pltpu.CompilerParams(collective_id=0),
    )(A, B)


# device time: 22280 ns/iter; 1.3954x vs baseline; 1.0161x over previous
import jax
import jax.numpy as jnp
from jax import lax
from jax.experimental import pallas as pl
from jax.experimental.pallas import tpu as pltpu

N_DEV = 16
P = 4
Q = 4


def kernel(A, B):
    m, k = A.shape
    _, n = B.shape
    blk = m // Q
    ch = blk // P

    def body(a_ref, b_ref, out_ref, part_ref, rs_ref, fch_ref,
             s1_sems, r1_sems, c1_s, c1_r, c2_s, c2_r):
        my = lax.axis_index("i")
        p = my // Q
        q = lax.rem(my, Q)
        own_row = q * blk + p * ch

        barrier = pltpu.get_barrier_semaphore()
        for s in range(1, N_DEV):
            dst = lax.rem(my + s, N_DEV)
            pl.semaphore_signal(
                barrier, inc=1, device_id=(dst,),
                device_id_type=pl.DeviceIdType.MESH,
            )
        a = a_ref[...].astype(jnp.bfloat16)
        b = b_ref[...].astype(jnp.bfloat16)
        part_ref[...] = jnp.dot(
            a, b, preferred_element_type=jnp.float32
        ).astype(jnp.bfloat16)
        pl.semaphore_wait(barrier, N_DEV - 1)

        p1 = []
        for s in range(1, N_DEV):
            dst = lax.rem(my + s, N_DEV)
            dst_row = lax.rem(dst, Q) * blk + (dst // Q) * ch
            rdma = pltpu.make_async_remote_copy(
                src_ref=part_ref.at[pl.ds(dst_row, ch), :],
                dst_ref=rs_ref.at[s],
                send_sem=s1_sems.at[s],
                recv_sem=r1_sems.at[s],
                device_id=(dst,),
                device_id_type=pl.DeviceIdType.MESH,
            )
            rdma.start()
            p1.append(rdma)
        acc = part_ref[pl.ds(own_row, ch), :].astype(jnp.float32)
        for s in range(1, N_DEV):
            p1[s - 1].wait_recv()
            acc = acc + rs_ref[s].astype(jnp.float32)
        final = jnp.maximum(acc, 0.0).astype(jnp.bfloat16)
        fch_ref[...] = final
        out_ref[pl.ds(own_row, ch), :] = final

        pC1 = []
        for s in range(1, P):
            pd = lax.rem(p + s, P)
            rdma = pltpu.make_async_remote_copy(
                src_ref=fch_ref,
                dst_ref=out_ref.at[pl.ds(own_row, ch), :],
                send_sem=c1_s.at[s],
                recv_sem=c1_r.at[s],
                device_id=(pd * Q + q,),
                device_id_type=pl.DeviceIdType.MESH,
            )
            rdma.start()
            pC1.append(rdma)
        for rdma in pC1:
            rdma.wait_recv()

        pC2 = []
        for s in range(1, Q):
            qd = lax.rem(q + s, Q)
            rdma = pltpu.make_async_remote_copy(
                src_ref=out_ref.at[pl.ds(q * blk, blk), :],
                dst_ref=out_ref.at[pl.ds(q * blk, blk), :],
                send_sem=c2_s.at[s],
                recv_sem=c2_r.at[s],
                device_id=(p * Q + qd,),
                device_id_type=pl.DeviceIdType.MESH,
            )
            rdma.start()
            pC2.append(rdma)
        for rdma in pC2:
            rdma.wait_recv()

        for rdma in p1 + pC1 + pC2:
            rdma.wait_send()

    return pl.pallas_call(
        body,
        out_shape=jax.ShapeDtypeStruct((m, n), jnp.bfloat16),
        in_specs=[
            pl.BlockSpec(memory_space=pltpu.VMEM),
            pl.BlockSpec(memory_space=pltpu.VMEM),
        ],
        out_specs=pl.BlockSpec(memory_space=pltpu.VMEM),
        scratch_shapes=[
            pltpu.VMEM((m, n), jnp.bfloat16),
            pltpu.VMEM((N_DEV, ch, n), jnp.bfloat16),
            pltpu.VMEM((ch, n), jnp.bfloat16),
            pltpu.SemaphoreType.DMA((N_DEV,)),
            pltpu.SemaphoreType.DMA((N_DEV,)),
            pltpu.SemaphoreType.DMA((P,)),
            pltpu.SemaphoreType.DMA((P,)),
            pltpu.SemaphoreType.DMA((Q,)),
            pltpu.SemaphoreType.DMA((Q,)),
        ],
        compiler_params=pltpu.CompilerParams(collective_id=0),
    )(A, B)


# device time: 20989 ns/iter; 1.4812x vs baseline; 1.0615x over previous
import jax
import jax.numpy as jnp
from jax import lax
from jax.experimental import pallas as pl
from jax.experimental.pallas import tpu as pltpu

N_DEV = 16
H = 2


def kernel(A, B):
    m, k = A.shape
    _, n = B.shape
    ch = m // N_DEV
    hr = ch // H

    def body(a_ref, b_ref, out_ref, part_ref, rs_ref, bc_ref,
             rs_s, rs_r, ag_s, ag_r):
        my = lax.axis_index("i")

        barrier = pltpu.get_barrier_semaphore()
        for s in range(1, N_DEV):
            dst = lax.rem(my + s, N_DEV)
            pl.semaphore_signal(
                barrier, inc=1, device_id=(dst,),
                device_id_type=pl.DeviceIdType.MESH,
            )
        a = a_ref[...].astype(jnp.bfloat16)
        b = b_ref[...].astype(jnp.bfloat16)
        part_ref[...] = jnp.dot(
            a, b, preferred_element_type=jnp.float32
        ).astype(jnp.bfloat16)
        pl.semaphore_wait(barrier, N_DEV - 1)

        rs = {}
        for h in range(H):
            for s in range(1, N_DEV):
                dst = lax.rem(my + s, N_DEV)
                rdma = pltpu.make_async_remote_copy(
                    src_ref=part_ref.at[pl.ds(dst * ch + h * hr, hr), :],
                    dst_ref=rs_ref.at[h, s],
                    send_sem=rs_s.at[h, s],
                    recv_sem=rs_r.at[h, s],
                    device_id=(dst,),
                    device_id_type=pl.DeviceIdType.MESH,
                )
                rdma.start()
                rs[h, s] = rdma

        ag = {}
        for h in range(H):
            acc = part_ref[
                pl.ds(my * ch + h * hr, hr), :
            ].astype(jnp.float32)
            for s in range(1, N_DEV):
                rs[h, s].wait_recv()
                acc = acc + rs_ref[h, s].astype(jnp.float32)
            final = jnp.maximum(acc, 0.0).astype(jnp.bfloat16)
            bc_ref[h] = final
            out_ref[pl.ds(my * ch + h * hr, hr), :] = final
            for s in range(1, N_DEV):
                dst = lax.rem(my + s, N_DEV)
                rdma = pltpu.make_async_remote_copy(
                    src_ref=bc_ref.at[h],
                    dst_ref=out_ref.at[pl.ds(my * ch + h * hr, hr), :],
                    send_sem=ag_s.at[h, s],
                    recv_sem=ag_r.at[h, s],
                    device_id=(dst,),
                    device_id_type=pl.DeviceIdType.MESH,
                )
                rdma.start()
                ag[h, s] = rdma

        for h in range(H):
            for s in range(1, N_DEV):
                ag[h, s].wait_recv()
        for rdma in list(rs.values()) + list(ag.values()):
            rdma.wait_send()

    return pl.pallas_call(
        body,
        out_shape=jax.ShapeDtypeStruct((m, n), jnp.bfloat16),
        in_specs=[
            pl.BlockSpec(memory_space=pltpu.VMEM),
            pl.BlockSpec(memory_space=pltpu.VMEM),
        ],
        out_specs=pl.BlockSpec(memory_space=pltpu.VMEM),
        scratch_shapes=[
            pltpu.VMEM((m, n), jnp.bfloat16),
            pltpu.VMEM((H, N_DEV, hr, n), jnp.bfloat16),
            pltpu.VMEM((H, hr, n), jnp.bfloat16),
            pltpu.SemaphoreType.DMA((H, N_DEV)),
            pltpu.SemaphoreType.DMA((H, N_DEV)),
            pltpu.SemaphoreType.DMA((H, N_DEV)),
            pltpu.SemaphoreType.DMA((H, N_DEV)),
        ],
        compiler_params=pltpu.CompilerParams(collective_id=0),
    )(A, B)
